# baseline (speedup 1.0000x reference)
import functools

import jax
import jax.numpy as jnp
from jax import lax
from jax.experimental import pallas as pl
from jax.experimental.pallas import tpu as pltpu
from jax.experimental.pallas import tpu_sc as plsc

NC = 2
NS = 16
NW = NC * NS
CH = 64
LANES = 16
ZR = 125

_sc_mesh = plsc.VectorSubcoreMesh(core_axis_name="c", subcore_axis_name="s")


def _fill_rows(ref, rows, width, value):
    @pl.loop(0, rows)
    def _(i):
        @pl.loop(0, width, step=LANES)
        def _(j):
            ref[i, pl.ds(j, LANES)] = jnp.full((LANES,), value, jnp.float32)


def _row_quota(n):
    q = (n // (NS * 8)) * 8
    rem = n - q * NS
    zch = 48 if q % 48 == 0 else (16 if q % 16 == 0 else 8)
    return q, rem, zch


def _deg_partials(dst3, n, pad):
    epw = dst3.shape[0] // NW
    nchunk = epw // CH
    q, rem, zch = _row_quota(n)
    DW = 4
    nq = nchunk // DW

    @functools.partial(
        pl.kernel,
        out_type=jax.ShapeDtypeStruct((NC * n,), jnp.float32),
        mesh=_sc_mesh,
        scratch_types=[
            pltpu.VMEM((CH,), jnp.int32),
            pltpu.VMEM((CH,), jnp.int32),
            pltpu.VMEM((CH,), jnp.int32),
            pltpu.VMEM((CH,), jnp.int32),
            pltpu.VMEM((CH,), jnp.float32),
            pltpu.VMEM((zch,), jnp.float32),
            pltpu.VMEM((q,), jnp.float32),
            pltpu.VMEM_SHARED((n + pad,), jnp.float32),
            pltpu.SemaphoreType.DMA,
            pltpu.SemaphoreType.DMA,
            pltpu.SemaphoreType.DMA,
            pltpu.SemaphoreType.DMA,
            pltpu.SemaphoreType.DMA,
            pltpu.SemaphoreType.DMA,
            pltpu.SemaphoreType.DMA,
            pltpu.SemaphoreType.DMA,
        ],
    )
    def deg_kernel(dst_hbm, out_hbm, i0, i1, i2, i3, ones_v, zbuf, rd_v, acc,
                   e0, e1, e2, e3, f0, f1, f2, f3):
        cid = lax.axis_index("c")
        sid = lax.axis_index("s")
        w = sid * NC + cid

        @pl.loop(0, CH, step=LANES)
        def _(j):
            ones_v[pl.ds(j, LANES)] = jnp.full((LANES,), 1.0, jnp.float32)

        @pl.loop(0, zch, step=LANES)
        def _(j):
            zbuf[pl.ds(j, LANES)] = jnp.zeros((LANES,), jnp.float32)

        @pl.loop(0, q // zch)
        def _(r):
            pltpu.sync_copy(zbuf, acc.at[pl.ds(sid * q + r * zch, zch)])
        if rem:
            @pl.when(sid == NS - 1)
            def _():
                pltpu.sync_copy(zbuf.at[pl.ds(0, rem)],
                                acc.at[pl.ds(NS * q, rem)])
        plsc.subcore_barrier()

        idxs = (i0, i1, i2, i3)
        isems = (e0, e1, e2, e3)
        ssems = (f0, f1, f2, f3)

        @pl.loop(0, nq)
        def _(p):
            g = p * DW

            @pl.when(p > 0)
            def _():
                for i in range(DW):
                    pltpu.make_async_copy(ones_v, acc.at[idxs[i]],
                                          ssems[i]).wait()

            cps = [pltpu.async_copy(
                       dst_hbm.at[pl.ds(w * epw + (g + i) * CH, CH)],
                       idxs[i], isems[i])
                   for i in range(DW)]
            for i in range(DW):
                cps[i].wait()
                pltpu.async_copy(ones_v, acc.at[idxs[i]], ssems[i], add=True)

        for i in range(DW):
            pltpu.make_async_copy(ones_v, acc.at[idxs[i]], ssems[i]).wait()

        plsc.subcore_barrier()

        pltpu.sync_copy(acc.at[pl.ds(sid * q, q)], rd_v)
        pltpu.sync_copy(rd_v, out_hbm.at[pl.ds(cid * n + sid * q, q)])
        if rem:
            @pl.when(sid == NS - 1)
            def _():
                pltpu.sync_copy(acc.at[pl.ds(NS * q, rem)],
                                rd_v.at[pl.ds(0, rem)])
                pltpu.sync_copy(rd_v.at[pl.ds(0, rem)],
                                out_hbm.at[pl.ds(cid * n + NS * q, rem)])

    return deg_kernel(dst3)


def _scatter_rows(y, src3, dst3, pad):
    n, h = y.shape
    epw = src3.shape[0] // NW
    nchunk = epw // CH
    q, rem, zch = _row_quota(n)
    W = 4
    nq = nchunk // W

    @functools.partial(
        pl.kernel,
        out_type=jax.ShapeDtypeStruct((NC, n, h), jnp.float32),
        mesh=_sc_mesh,
        scratch_types=[
            pltpu.VMEM((epw,), jnp.int32),
            pltpu.VMEM((CH,), jnp.int32),
            pltpu.VMEM((CH,), jnp.int32),
            pltpu.VMEM((CH,), jnp.int32),
            pltpu.VMEM((CH,), jnp.int32),
            pltpu.VMEM((CH, h), jnp.float32),
            pltpu.VMEM((CH, h), jnp.float32),
            pltpu.VMEM((CH, h), jnp.float32),
            pltpu.VMEM((CH, h), jnp.float32),
            pltpu.VMEM((zch, h), jnp.float32),
            pltpu.VMEM_SHARED((n + pad, h), jnp.float32),
            pltpu.SemaphoreType.DMA,
            pltpu.SemaphoreType.DMA,
            pltpu.SemaphoreType.DMA,
            pltpu.SemaphoreType.DMA,
            pltpu.SemaphoreType.DMA,
            pltpu.SemaphoreType.DMA,
            pltpu.SemaphoreType.DMA,
            pltpu.SemaphoreType.DMA,
            pltpu.SemaphoreType.DMA,
            pltpu.SemaphoreType.DMA,
            pltpu.SemaphoreType.DMA,
            pltpu.SemaphoreType.DMA,
        ],
    )
    def scatter_kernel(y_hbm, src_hbm, dst_hbm, out_hbm,
                       src_v, d0, d1, d2, d3, r0, r1, r2, r3, zbuf, acc,
                       s0, s1, s2, s3, e0, e1, e2, e3, f0, f1, f2, f3):
        cid = lax.axis_index("c")
        sid = lax.axis_index("s")
        w = sid * NC + cid

        slab = pltpu.async_copy(src_hbm.at[pl.ds(w * epw, epw)], src_v, s0)

        _fill_rows(zbuf, zch, h, 0.0)

        zcps = [pltpu.async_copy(zbuf, acc.at[pl.ds(sid * q + r * zch, zch)],
                                 s1)
                for r in range(q // zch)]
        if rem:
            @pl.when(sid == NS - 1)
            def _():
                pltpu.sync_copy(zbuf.at[pl.ds(0, rem)],
                                acc.at[pl.ds(NS * q, rem)])
        for cp in zcps:
            cp.wait()
        slab.wait()
        plsc.subcore_barrier()

        rows = (r0, r1, r2, r3)
        sems = (s0, s1, s2, s3)
        idxs = (d0, d1, d2, d3)
        isems = (e0, e1, e2, e3)
        ssems = (f0, f1, f2, f3)

        @pl.loop(0, nq)
        def _(p):
            g = p * W

            @pl.when(p > 0)
            def _():
                for i in range(W):
                    pltpu.make_async_copy(rows[i], acc.at[idxs[i]],
                                          ssems[i]).wait()

            ci = [pltpu.async_copy(
                      dst_hbm.at[pl.ds(w * epw + (g + i) * CH, CH)],
                      idxs[i], isems[i])
                  for i in range(W)]
            cg = [pltpu.async_copy(
                      y_hbm.at[src_v.at[pl.ds((g + i) * CH, CH)]],
                      rows[i], sems[i])
                  for i in range(W)]
            for i in range(W):
                ci[i].wait()
                cg[i].wait()
                pltpu.async_copy(rows[i], acc.at[idxs[i]], ssems[i], add=True)

        for i in range(W):
            pltpu.make_async_copy(rows[i], acc.at[idxs[i]], ssems[i]).wait()

        plsc.subcore_barrier()

        pltpu.sync_copy(acc.at[pl.ds(sid * q, q)],
                        out_hbm.at[cid, pl.ds(sid * q, q)])
        if rem:
            @pl.when(sid == NS - 1)
            def _():
                pltpu.sync_copy(acc.at[pl.ds(NS * q, rem)],
                                out_hbm.at[cid, pl.ds(NS * q, rem)])

    return scatter_kernel(y, src3, dst3)


def _norm_from(deg_ref):
    d = deg_ref[:, 0:1] + deg_ref[:, 1:2]
    return lax.rsqrt(jnp.maximum(d, 1.0))


def _mm_scale(x, w, degp, block):
    n, d = x.shape
    h = w.shape[1]

    def body(deg_ref, x_ref, w_ref, o_ref):
        nrm = _norm_from(deg_ref)
        o_ref[...] = jnp.dot(x_ref[...], w_ref[...],
                             preferred_element_type=jnp.float32) * nrm

    return pl.pallas_call(
        body,
        grid=(n // block,),
        in_specs=[
            pl.BlockSpec((block, 2), lambda i: (i, 0)),
            pl.BlockSpec((block, d), lambda i: (i, 0)),
            pl.BlockSpec((d, h), lambda i: (0, 0)),
        ],
        out_specs=pl.BlockSpec((block, h), lambda i: (i, 0)),
        out_shape=jax.ShapeDtypeStruct((n, h), jnp.float32),
    )(degp, x, w)


def _layer_mid(s, degp, w, b, block):
    _, n, h = s.shape

    def body(deg_ref, s_ref, w_ref, b_ref, o_ref):
        nrm = _norm_from(deg_ref)
        hid = jnp.maximum((s_ref[0] + s_ref[1]) * nrm + b_ref[...], 0.0)
        o_ref[...] = jnp.dot(hid, w_ref[...],
                             preferred_element_type=jnp.float32) * nrm

    return pl.pallas_call(
        body,
        grid=(n // block,),
        in_specs=[
            pl.BlockSpec((block, 2), lambda i: (i, 0)),
            pl.BlockSpec((2, block, h), lambda i: (0, i, 0)),
            pl.BlockSpec((h, h), lambda i: (0, 0)),
            pl.BlockSpec((1, h), lambda i: (0, 0)),
        ],
        out_specs=pl.BlockSpec((block, h), lambda i: (i, 0)),
        out_shape=jax.ShapeDtypeStruct((n, h), jnp.float32),
    )(degp, s, w, b)


def _readout(s, degp, b, wg, bg, block):
    _, n, h = s.shape

    def body(deg_ref, s_ref, b_ref, wg_ref, bg_ref, h_ref, wh_ref):
        nrm = _norm_from(deg_ref)
        hid = jnp.maximum((s_ref[0] + s_ref[1]) * nrm + b_ref[...], 0.0)
        h_ref[...] = hid
        gate = jax.nn.sigmoid(
            jnp.dot(hid, wg_ref[...], preferred_element_type=jnp.float32)
            + bg_ref[...])
        part = jnp.sum(gate * hid, axis=0, keepdims=True)

        @pl.when(pl.program_id(0) == 0)
        def _():
            wh_ref[...] = jnp.zeros_like(wh_ref)

        wh_ref[...] += part

    return pl.pallas_call(
        body,
        grid=(n // block,),
        in_specs=[
            pl.BlockSpec((block, 2), lambda i: (i, 0)),
            pl.BlockSpec((2, block, h), lambda i: (0, i, 0)),
            pl.BlockSpec((1, h), lambda i: (0, 0)),
            pl.BlockSpec((h, 1), lambda i: (0, 0)),
            pl.BlockSpec((1, 1), lambda i: (0, 0)),
        ],
        out_specs=[
            pl.BlockSpec((block, h), lambda i: (i, 0)),
            pl.BlockSpec((1, h), lambda i: (0, 0)),
        ],
        out_shape=[
            jax.ShapeDtypeStruct((n, h), jnp.float32),
            jax.ShapeDtypeStruct((1, h), jnp.float32),
        ],
    )(degp, s, b, wg, bg)


def kernel(x, edge_index, W1, b1, W2, b2, Wg, bg):
    n, d = x.shape
    h = W1.shape[1]
    e = edge_index.shape[1]
    block = 1000

    pad = 8
    epw = e // NW
    epw_pad = ((epw + CH - 1) // CH) * CH
    while (epw_pad // CH) % 4:
        epw_pad += CH
    npad = epw_pad - epw
    j = jnp.arange(npad, dtype=jnp.int32)
    src_fill = jnp.broadcast_to((j * 997) % n, (NW, npad))
    dst_fill = jnp.broadcast_to(n + (j % pad), (NW, npad))
    src3 = jnp.concatenate(
        [edge_index[0].reshape(NW, epw), src_fill], axis=1).reshape(-1)
    dst3 = jnp.concatenate(
        [edge_index[1].reshape(NW, epw), dst_fill], axis=1).reshape(-1)

    degp = _deg_partials(dst3, n, pad).reshape(NC, n).T

    y1 = _mm_scale(x, W1, degp, block)
    s1 = _scatter_rows(y1, src3, dst3, pad)
    y2 = _layer_mid(s1, degp, W2, b1.reshape(1, h), block)
    s2 = _scatter_rows(y2, src3, dst3, pad)
    h2, wh = _readout(s2, degp, b2.reshape(1, h), Wg, bg.reshape(1, 1), block)
    return wh.reshape(h), h2


# speedup vs baseline: 1.0226x; 1.0226x over previous
import functools

import jax
import jax.numpy as jnp
from jax import lax
from jax.experimental import pallas as pl
from jax.experimental.pallas import tpu as pltpu
from jax.experimental.pallas import tpu_sc as plsc

NC = 2
NS = 16
NW = NC * NS
CH = 128
LANES = 16
ZR = 125

_sc_mesh = plsc.VectorSubcoreMesh(core_axis_name="c", subcore_axis_name="s")


def _fill_rows(ref, rows, width, value):
    @pl.loop(0, rows)
    def _(i):
        @pl.loop(0, width, step=LANES)
        def _(j):
            ref[i, pl.ds(j, LANES)] = jnp.full((LANES,), value, jnp.float32)


def _row_quota(n):
    q = (n // (NS * 8)) * 8
    rem = n - q * NS
    zch = 48 if q % 48 == 0 else (16 if q % 16 == 0 else 8)
    return q, rem, zch


def _deg_partials(dst3, n, pad):
    epw = dst3.shape[0] // NW
    nchunk = epw // CH
    q, rem, zch = _row_quota(n)
    DW = 4
    nq = nchunk // DW

    @functools.partial(
        pl.kernel,
        out_type=jax.ShapeDtypeStruct((NC * n,), jnp.float32),
        mesh=_sc_mesh,
        scratch_types=[
            pltpu.VMEM((CH,), jnp.int32),
            pltpu.VMEM((CH,), jnp.int32),
            pltpu.VMEM((CH,), jnp.int32),
            pltpu.VMEM((CH,), jnp.int32),
            pltpu.VMEM((CH,), jnp.float32),
            pltpu.VMEM((zch,), jnp.float32),
            pltpu.VMEM((q,), jnp.float32),
            pltpu.VMEM_SHARED((n + pad,), jnp.float32),
            pltpu.SemaphoreType.DMA,
            pltpu.SemaphoreType.DMA,
            pltpu.SemaphoreType.DMA,
            pltpu.SemaphoreType.DMA,
            pltpu.SemaphoreType.DMA,
            pltpu.SemaphoreType.DMA,
            pltpu.SemaphoreType.DMA,
            pltpu.SemaphoreType.DMA,
        ],
    )
    def deg_kernel(dst_hbm, out_hbm, i0, i1, i2, i3, ones_v, zbuf, rd_v, acc,
                   e0, e1, e2, e3, f0, f1, f2, f3):
        cid = lax.axis_index("c")
        sid = lax.axis_index("s")
        w = sid * NC + cid

        @pl.loop(0, CH, step=LANES)
        def _(j):
            ones_v[pl.ds(j, LANES)] = jnp.full((LANES,), 1.0, jnp.float32)

        @pl.loop(0, zch, step=LANES)
        def _(j):
            zbuf[pl.ds(j, LANES)] = jnp.zeros((LANES,), jnp.float32)

        @pl.loop(0, q // zch)
        def _(r):
            pltpu.sync_copy(zbuf, acc.at[pl.ds(sid * q + r * zch, zch)])
        if rem:
            @pl.when(sid == NS - 1)
            def _():
                pltpu.sync_copy(zbuf.at[pl.ds(0, rem)],
                                acc.at[pl.ds(NS * q, rem)])
        plsc.subcore_barrier()

        idxs = (i0, i1, i2, i3)
        isems = (e0, e1, e2, e3)
        ssems = (f0, f1, f2, f3)

        @pl.loop(0, nq)
        def _(p):
            g = p * DW

            @pl.when(p > 0)
            def _():
                for i in range(DW):
                    pltpu.make_async_copy(ones_v, acc.at[idxs[i]],
                                          ssems[i]).wait()

            cps = [pltpu.async_copy(
                       dst_hbm.at[pl.ds(w * epw + (g + i) * CH, CH)],
                       idxs[i], isems[i])
                   for i in range(DW)]
            for i in range(DW):
                cps[i].wait()
                pltpu.async_copy(ones_v, acc.at[idxs[i]], ssems[i], add=True)

        for i in range(DW):
            pltpu.make_async_copy(ones_v, acc.at[idxs[i]], ssems[i]).wait()

        plsc.subcore_barrier()

        pltpu.sync_copy(acc.at[pl.ds(sid * q, q)], rd_v)
        pltpu.sync_copy(rd_v, out_hbm.at[pl.ds(cid * n + sid * q, q)])
        if rem:
            @pl.when(sid == NS - 1)
            def _():
                pltpu.sync_copy(acc.at[pl.ds(NS * q, rem)],
                                rd_v.at[pl.ds(0, rem)])
                pltpu.sync_copy(rd_v.at[pl.ds(0, rem)],
                                out_hbm.at[pl.ds(cid * n + NS * q, rem)])

    return deg_kernel(dst3)


def _scatter_rows(y, src3, dst3, pad):
    n, h = y.shape
    epw = src3.shape[0] // NW
    nchunk = epw // CH
    q, rem, zch = _row_quota(n)
    W = 2
    nq = nchunk // W

    @functools.partial(
        pl.kernel,
        out_type=jax.ShapeDtypeStruct((NC, n, h), jnp.float32),
        mesh=_sc_mesh,
        scratch_types=[
            pltpu.VMEM((epw,), jnp.int32),
            pltpu.VMEM((CH,), jnp.int32),
            pltpu.VMEM((CH,), jnp.int32),
            pltpu.VMEM((CH, h), jnp.float32),
            pltpu.VMEM((CH, h), jnp.float32),
            pltpu.VMEM((zch, h), jnp.float32),
            pltpu.VMEM_SHARED((n + pad, h), jnp.float32),
            pltpu.SemaphoreType.DMA,
            pltpu.SemaphoreType.DMA,
            pltpu.SemaphoreType.DMA,
            pltpu.SemaphoreType.DMA,
            pltpu.SemaphoreType.DMA,
            pltpu.SemaphoreType.DMA,
        ],
    )
    def scatter_kernel(y_hbm, src_hbm, dst_hbm, out_hbm,
                       src_v, d0, d1, r0, r1, zbuf, acc,
                       s0, s1, e0, e1, f0, f1):
        cid = lax.axis_index("c")
        sid = lax.axis_index("s")
        w = sid * NC + cid

        slab = pltpu.async_copy(src_hbm.at[pl.ds(w * epw, epw)], src_v, s0)

        _fill_rows(zbuf, zch, h, 0.0)

        zcps = [pltpu.async_copy(zbuf, acc.at[pl.ds(sid * q + r * zch, zch)],
                                 s1)
                for r in range(q // zch)]
        if rem:
            @pl.when(sid == NS - 1)
            def _():
                pltpu.sync_copy(zbuf.at[pl.ds(0, rem)],
                                acc.at[pl.ds(NS * q, rem)])
        for cp in zcps:
            cp.wait()
        slab.wait()
        plsc.subcore_barrier()

        rows = (r0, r1)
        sems = (s0, s1)
        idxs = (d0, d1)
        isems = (e0, e1)
        ssems = (f0, f1)

        @pl.loop(0, nq)
        def _(p):
            g = p * W

            @pl.when(p > 0)
            def _():
                for i in range(W):
                    pltpu.make_async_copy(rows[i], acc.at[idxs[i]],
                                          ssems[i]).wait()

            ci = [pltpu.async_copy(
                      dst_hbm.at[pl.ds(w * epw + (g + i) * CH, CH)],
                      idxs[i], isems[i])
                  for i in range(W)]
            cg = [pltpu.async_copy(
                      y_hbm.at[src_v.at[pl.ds((g + i) * CH, CH)]],
                      rows[i], sems[i])
                  for i in range(W)]
            for i in range(W):
                ci[i].wait()
                cg[i].wait()
                pltpu.async_copy(rows[i], acc.at[idxs[i]], ssems[i], add=True)

        for i in range(W):
            pltpu.make_async_copy(rows[i], acc.at[idxs[i]], ssems[i]).wait()

        plsc.subcore_barrier()

        pltpu.sync_copy(acc.at[pl.ds(sid * q, q)],
                        out_hbm.at[cid, pl.ds(sid * q, q)])
        if rem:
            @pl.when(sid == NS - 1)
            def _():
                pltpu.sync_copy(acc.at[pl.ds(NS * q, rem)],
                                out_hbm.at[cid, pl.ds(NS * q, rem)])

    return scatter_kernel(y, src3, dst3)


def _norm_from(deg_ref):
    d = deg_ref[:, 0:1] + deg_ref[:, 1:2]
    return lax.rsqrt(jnp.maximum(d, 1.0))


def _mm_scale(x, w, degp, block):
    n, d = x.shape
    h = w.shape[1]

    def body(deg_ref, x_ref, w_ref, o_ref):
        nrm = _norm_from(deg_ref)
        o_ref[...] = jnp.dot(x_ref[...], w_ref[...],
                             preferred_element_type=jnp.float32) * nrm

    return pl.pallas_call(
        body,
        grid=(n // block,),
        in_specs=[
            pl.BlockSpec((block, 2), lambda i: (i, 0)),
            pl.BlockSpec((block, d), lambda i: (i, 0)),
            pl.BlockSpec((d, h), lambda i: (0, 0)),
        ],
        out_specs=pl.BlockSpec((block, h), lambda i: (i, 0)),
        out_shape=jax.ShapeDtypeStruct((n, h), jnp.float32),
    )(degp, x, w)


def _layer_mid(s, degp, w, b, block):
    _, n, h = s.shape

    def body(deg_ref, s_ref, w_ref, b_ref, o_ref):
        nrm = _norm_from(deg_ref)
        hid = jnp.maximum((s_ref[0] + s_ref[1]) * nrm + b_ref[...], 0.0)
        o_ref[...] = jnp.dot(hid, w_ref[...],
                             preferred_element_type=jnp.float32) * nrm

    return pl.pallas_call(
        body,
        grid=(n // block,),
        in_specs=[
            pl.BlockSpec((block, 2), lambda i: (i, 0)),
            pl.BlockSpec((2, block, h), lambda i: (0, i, 0)),
            pl.BlockSpec((h, h), lambda i: (0, 0)),
            pl.BlockSpec((1, h), lambda i: (0, 0)),
        ],
        out_specs=pl.BlockSpec((block, h), lambda i: (i, 0)),
        out_shape=jax.ShapeDtypeStruct((n, h), jnp.float32),
    )(degp, s, w, b)


def _readout(s, degp, b, wg, bg, block):
    _, n, h = s.shape

    def body(deg_ref, s_ref, b_ref, wg_ref, bg_ref, h_ref, wh_ref):
        nrm = _norm_from(deg_ref)
        hid = jnp.maximum((s_ref[0] + s_ref[1]) * nrm + b_ref[...], 0.0)
        h_ref[...] = hid
        gate = jax.nn.sigmoid(
            jnp.dot(hid, wg_ref[...], preferred_element_type=jnp.float32)
            + bg_ref[...])
        part = jnp.sum(gate * hid, axis=0, keepdims=True)

        @pl.when(pl.program_id(0) == 0)
        def _():
            wh_ref[...] = jnp.zeros_like(wh_ref)

        wh_ref[...] += part

    return pl.pallas_call(
        body,
        grid=(n // block,),
        in_specs=[
            pl.BlockSpec((block, 2), lambda i: (i, 0)),
            pl.BlockSpec((2, block, h), lambda i: (0, i, 0)),
            pl.BlockSpec((1, h), lambda i: (0, 0)),
            pl.BlockSpec((h, 1), lambda i: (0, 0)),
            pl.BlockSpec((1, 1), lambda i: (0, 0)),
        ],
        out_specs=[
            pl.BlockSpec((block, h), lambda i: (i, 0)),
            pl.BlockSpec((1, h), lambda i: (0, 0)),
        ],
        out_shape=[
            jax.ShapeDtypeStruct((n, h), jnp.float32),
            jax.ShapeDtypeStruct((1, h), jnp.float32),
        ],
    )(degp, s, b, wg, bg)


def kernel(x, edge_index, W1, b1, W2, b2, Wg, bg):
    n, d = x.shape
    h = W1.shape[1]
    e = edge_index.shape[1]
    block = 1000

    pad = 8
    epw = e // NW
    epw_pad = ((epw + CH - 1) // CH) * CH
    while (epw_pad // CH) % 4:
        epw_pad += CH
    npad = epw_pad - epw
    j = jnp.arange(npad, dtype=jnp.int32)
    src_fill = jnp.broadcast_to((j * 997) % n, (NW, npad))
    dst_fill = jnp.broadcast_to(n + (j % pad), (NW, npad))
    src3 = jnp.concatenate(
        [edge_index[0].reshape(NW, epw), src_fill], axis=1).reshape(-1)
    dst3 = jnp.concatenate(
        [edge_index[1].reshape(NW, epw), dst_fill], axis=1).reshape(-1)

    degp = _deg_partials(dst3, n, pad).reshape(NC, n).T

    y1 = _mm_scale(x, W1, degp, block)
    s1 = _scatter_rows(y1, src3, dst3, pad)
    y2 = _layer_mid(s1, degp, W2, b1.reshape(1, h), block)
    s2 = _scatter_rows(y2, src3, dst3, pad)
    h2, wh = _readout(s2, degp, b2.reshape(1, h), Wg, bg.reshape(1, 1), block)
    return wh.reshape(h), h2
